# Initial kernel scaffold; baseline (speedup 1.0000x reference)
import jax
import jax.numpy as jnp
from jax.experimental import pallas as pl


def kernel(continuous_features, categorical_features, tables, Wc, bc, W1, b1, W2, b2, Wo, bo):
    raise NotImplementedError("write your pallas kernel here")


# speedup vs baseline: 2.2059x; 2.2059x over previous
import functools

import jax
import jax.numpy as jnp
from jax import lax
from jax.experimental import pallas as pl
from jax.experimental.pallas import tpu as pltpu
from jax.experimental.pallas import tpu_sc as plsc

B = 4096
F = 13
NF = 26
V = 100000
D = 32
H1 = 512
H2 = 256
MLP_IN = D + NF * D

_NC = 2
_NS = 16
_NW = _NC * _NS


def _sc_gather(tables2d, flat_idx):
    n = flat_idx.shape[0]
    per_w = n // _NW
    mesh = plsc.VectorSubcoreMesh(core_axis_name="c", subcore_axis_name="s")

    @functools.partial(
        pl.kernel,
        mesh=mesh,
        compiler_params=pltpu.CompilerParams(use_tc_tiling_on_sc=False),
        out_type=jax.ShapeDtypeStruct((n, D), jnp.float32),
        scratch_types=[
            pltpu.VMEM((per_w,), jnp.int32),
            pltpu.VMEM((per_w, D), jnp.float32),
            pltpu.SemaphoreType.DMA,
        ],
    )
    def k(table_hbm, idx_hbm, out_hbm, idx_v, rows_v, sem):
        wid = lax.axis_index("s") * _NC + lax.axis_index("c")
        base = wid * per_w
        pltpu.sync_copy(idx_hbm.at[pl.ds(base, per_w)], idx_v)
        pltpu.async_copy(table_hbm.at[idx_v], rows_v, sem).wait()
        pltpu.sync_copy(rows_v, out_hbm.at[pl.ds(base, per_w)])

    return k(tables2d, flat_idx)


def _mlp_body(cont_ref, emb_ref, Wc_ref, bc_ref, W1_ref, b1_ref, W2_ref,
              b2_ref, Wo_ref, bo_ref, out_ref):
    xc = jnp.dot(cont_ref[...], Wc_ref[...],
                 preferred_element_type=jnp.float32) + bc_ref[...]
    x = jnp.concatenate([xc, emb_ref[...]], axis=1)
    h1 = jnp.maximum(
        jnp.dot(x, W1_ref[...], preferred_element_type=jnp.float32)
        + b1_ref[...], 0.0)
    h2 = jnp.maximum(
        jnp.dot(h1, W2_ref[...], preferred_element_type=jnp.float32)
        + b2_ref[...], 0.0)
    o = jnp.dot(h2, Wo_ref[...], preferred_element_type=jnp.float32) + bo_ref[...]
    out_ref[...] = jax.nn.sigmoid(o)


def _tc_mlp(cont, emb2d, Wc, bc, W1, b1, W2, b2, Wo, bo):
    blk = 512
    grid = (B // blk,)
    return pl.pallas_call(
        _mlp_body,
        grid=grid,
        in_specs=[
            pl.BlockSpec((blk, F), lambda i: (i, 0)),
            pl.BlockSpec((blk, NF * D), lambda i: (i, 0)),
            pl.BlockSpec((F, D), lambda i: (0, 0)),
            pl.BlockSpec((1, D), lambda i: (0, 0)),
            pl.BlockSpec((MLP_IN, H1), lambda i: (0, 0)),
            pl.BlockSpec((1, H1), lambda i: (0, 0)),
            pl.BlockSpec((H1, H2), lambda i: (0, 0)),
            pl.BlockSpec((1, H2), lambda i: (0, 0)),
            pl.BlockSpec((H2, 1), lambda i: (0, 0)),
            pl.BlockSpec((1, 1), lambda i: (0, 0)),
        ],
        out_specs=pl.BlockSpec((blk, 1), lambda i: (i, 0)),
        out_shape=jax.ShapeDtypeStruct((B, 1), jnp.float32),
    )(cont, emb2d, Wc, bc, W1, b1, W2, b2, Wo, bo)


def kernel(continuous_features, categorical_features, tables, Wc, bc, W1, b1,
           W2, b2, Wo, bo):
    tables2d = tables.reshape(NF * V, D)
    offsets = (jnp.arange(NF, dtype=jnp.int32) * V)[None, :]
    flat_idx = (categorical_features.astype(jnp.int32) + offsets).reshape(B * NF)
    emb = _sc_gather(tables2d, flat_idx)
    emb2d = emb.reshape(B, NF * D)
    return _tc_mlp(continuous_features, emb2d,
                   Wc, bc.reshape(1, D),
                   W1, b1.reshape(1, H1),
                   W2, b2.reshape(1, H2),
                   Wo, bo.reshape(1, 1))
